# baseline (speedup 1.0000x reference)
import functools

import jax
import jax.numpy as jnp
from jax import lax
from jax.experimental import pallas as pl
from jax.experimental.pallas import tpu as pltpu
from jax.experimental.pallas import tpu_sc as plsc

_NB = 2


def _make_tc_padder(V, N):
    BLK = 512

    def body(tt_ref, o_ref):
        t = jax.lax.dot_general(
            tt_ref[...],
            jnp.eye(N, dtype=jnp.float32),
            (((0,), (0,)), ((), ())),
            preferred_element_type=jnp.float32,
        )
        o_ref[...] = jnp.pad(t, ((0, 0), (0, 128 - N)))

    return pl.pallas_call(
        body,
        grid=(pl.cdiv(V, BLK),),
        in_specs=[pl.BlockSpec((N, BLK), lambda i: (0, i))],
        out_specs=pl.BlockSpec((BLK, 128), lambda i: (i, 0)),
        out_shape=jax.ShapeDtypeStruct((V, 128), jnp.float32),
    )


def _make_sc_kernel(B, L, W, N, NC, NS):
    NW = NC * NS
    BM = 64
    NBLK = B // 128
    n_chunks_tot = L * NBLK * 2
    n_chunks = n_chunks_tot // NW
    n_rounds = n_chunks // _NB
    NR = N // 8
    LANES = 16
    inv_w = 1.0 / W
    idx_rows_per_w = (n_chunks // 2) // 2

    mesh = plsc.VectorSubcoreMesh(
        core_axis_name="c", subcore_axis_name="s", num_cores=NC, num_subcores=NS
    )

    @functools.partial(
        pl.kernel,
        out_type=jax.ShapeDtypeStruct((L, NR, NBLK, 8, 128), jnp.float32),
        mesh=mesh,
        scratch_types=[
            pltpu.VMEM((idx_rows_per_w, 8, 128), jnp.int32),
            pltpu.VMEM((_NB, W, BM, 128), jnp.float32),
            pltpu.VMEM((_NB, NR, 8, 128), jnp.float32),
        ]
        + [pltpu.SemaphoreType.DMA] * (2 * _NB),
        compiler_params=pltpu.CompilerParams(
            needs_layout_passes=False, disable_bounds_checks=True
        ),
    )
    def sc_kernel(idx_hbm, table_hbm, out_hbm, idx_v, rows_v, out_v, *sems):
        gsems, ssems = sems[:_NB], sems[_NB:]
        wid = lax.axis_index("s") * NC + lax.axis_index("c")
        pltpu.sync_copy(
            idx_hbm.at[pl.ds(wid * idx_rows_per_w, idx_rows_per_w)], idx_v
        )

        def idx_slice(t, w):
            h = t & 1
            lcl = t >> 1
            p = lcl >> 1
            j = ((lcl & 1) << 2) + w
            return idx_v.at[p, j, pl.ds(h * BM, BM)]

        def fire_gathers(t, b):
            for w in range(W):
                pltpu.async_copy(
                    table_hbm.at[idx_slice(t, w)], rows_v.at[b, w], gsems[b]
                )

        def wait_gathers(t, b):
            for w in range(W):
                pltpu.make_async_copy(
                    table_hbm.at[idx_slice(t, w)], rows_v.at[b, w], gsems[b]
                ).wait()

        def out_block(t):
            g = wid * n_chunks + t
            l = g // (NBLK * 2)
            rem = g % (NBLK * 2)
            c = rem >> 1
            h = rem & 1
            return out_hbm.at[l, pl.ds(0, NR), c, pl.ds(0, 8), pl.ds(h * BM, BM)]

        def out_src(b):
            return out_v.at[b, pl.ds(0, NR), pl.ds(0, 8), pl.ds(0, BM)]

        for b in range(_NB):
            fire_gathers(b, b)

        iota = lax.iota(jnp.int32, LANES)
        nr_vecs = [(iota + n0 * LANES) >> 3 for n0 in range(N // LANES)]
        n8_vecs = [(iota + n0 * LANES) & 7 for n0 in range(N // LANES)]

        def round_body(r, carry):
            for b in range(_NB):
                t = r * _NB + b
                wait_gathers(t, b)

                @pl.when(r > 0)
                def _():
                    pltpu.make_async_copy(
                        out_src(b), out_block(t - _NB), ssems[b]
                    ).wait()

                def bm_body(bm, _):
                    bm_vec = jnp.full((LANES,), 0, jnp.int32) + bm
                    for n0 in range(N // LANES):
                        sl = pl.ds(n0 * LANES, LANES)
                        acc = rows_v[b, 0, bm, sl]
                        for w in range(1, W):
                            acc = acc + rows_v[b, w, bm, sl]
                        plsc.store_scatter(
                            out_v.at[b],
                            [nr_vecs[n0], n8_vecs[n0], bm_vec],
                            acc * inv_w,
                        )
                    return _

                lax.fori_loop(0, BM, bm_body, 0)

                pltpu.async_copy(out_src(b), out_block(t), ssems[b])

                @pl.when(t + _NB < n_chunks)
                def _():
                    fire_gathers(t + _NB, b)

            return carry

        lax.fori_loop(0, n_rounds, round_body, 0)

        for b in range(_NB):
            pltpu.make_async_copy(
                out_src(b), out_block(n_chunks - _NB + b), ssems[b]
            ).wait()

    return sc_kernel


def kernel(indices, table):
    B, L, W = indices.shape
    V, N = table.shape
    try:
        info = plsc.get_sparse_core_info()
        NC, NS = info.num_cores, info.num_subcores
    except ValueError:
        NC, NS = 2, 16
    if indices.dtype != jnp.int32:
        indices = indices.astype(jnp.int32)
    tpad = jnp.pad(table, ((0, 0), (0, 128 - N)))
    it = jnp.transpose(indices, (1, 2, 0))
    it = it.reshape(L, W, B // 128, 128)
    it = jnp.transpose(it, (0, 2, 1, 3))
    it = it.reshape(L * B // 256, 8, 128)
    out = _make_sc_kernel(B, L, W, N, NC, NS)(it, tpad)
    o = jnp.transpose(out, (0, 1, 3, 2, 4))
    o = o.reshape(L, N, B)
    return jnp.transpose(o, (2, 0, 1))


# speedup vs baseline: 1.0010x; 1.0010x over previous
import functools

import jax
import jax.numpy as jnp
from jax import lax
from jax.experimental import pallas as pl
from jax.experimental.pallas import tpu as pltpu
from jax.experimental.pallas import tpu_sc as plsc

_NB = 2


def _make_tc_padder(V, N):
    BLK = 512

    def body(tt_ref, o_ref):
        t = jax.lax.dot_general(
            tt_ref[...],
            jnp.eye(N, dtype=jnp.float32),
            (((0,), (0,)), ((), ())),
            preferred_element_type=jnp.float32,
        )
        o_ref[...] = jnp.pad(t, ((0, 0), (0, 128 - N)))

    return pl.pallas_call(
        body,
        grid=(pl.cdiv(V, BLK),),
        in_specs=[pl.BlockSpec((N, BLK), lambda i: (0, i))],
        out_specs=pl.BlockSpec((BLK, 128), lambda i: (i, 0)),
        out_shape=jax.ShapeDtypeStruct((V, 128), jnp.float32),
    )


def _make_sc_kernel(B, L, W, N, NC, NS):
    NW = NC * NS
    BM = 64
    NBLK = B // 128
    n_chunks_tot = L * NBLK * 2
    n_chunks = n_chunks_tot // NW
    n_rounds = n_chunks // _NB
    NR = N // 8
    LANES = 16
    inv_w = 1.0 / W
    idx_rows_per_w = (n_chunks // 2) // 2

    mesh = plsc.VectorSubcoreMesh(
        core_axis_name="c", subcore_axis_name="s", num_cores=NC, num_subcores=NS
    )

    @functools.partial(
        pl.kernel,
        out_type=jax.ShapeDtypeStruct((L, NR, NBLK, 8, 128), jnp.float32),
        mesh=mesh,
        scratch_types=[
            pltpu.VMEM((idx_rows_per_w, 8, 128), jnp.int32),
            pltpu.VMEM((_NB, W, BM, 128), jnp.float32),
            pltpu.VMEM((_NB, NR, 8, 128), jnp.float32),
        ]
        + [pltpu.SemaphoreType.DMA] * (2 * _NB),
        compiler_params=pltpu.CompilerParams(
            needs_layout_passes=False, disable_bounds_checks=True
        ),
    )
    def sc_kernel(idx_hbm, table_hbm, out_hbm, idx_v, rows_v, out_v, *sems):
        gsems, ssems = sems[:_NB], sems[_NB:]
        wid = lax.axis_index("s") * NC + lax.axis_index("c")
        pltpu.sync_copy(
            idx_hbm.at[pl.ds(wid * idx_rows_per_w, idx_rows_per_w)], idx_v
        )

        def idx_slice(t, w):
            h = t & 1
            lcl = t >> 1
            p = lcl >> 1
            j = ((lcl & 1) << 2) + w
            return idx_v.at[p, j, pl.ds(h * BM, BM)]

        def fire_gathers(t, b):
            for w in range(W):
                pltpu.async_copy(
                    table_hbm.at[idx_slice(t, w)], rows_v.at[b, w], gsems[b]
                )

        def wait_gathers(t, b):
            for w in range(W):
                pltpu.make_async_copy(
                    table_hbm.at[idx_slice(t, w)], rows_v.at[b, w], gsems[b]
                ).wait()

        def out_block(t):
            g = wid * n_chunks + t
            l = g // (NBLK * 2)
            rem = g % (NBLK * 2)
            c = rem >> 1
            h = rem & 1
            return out_hbm.at[l, pl.ds(0, NR), c, pl.ds(0, 8), pl.ds(h * BM, BM)]

        def out_src(b):
            return out_v.at[b, pl.ds(0, NR), pl.ds(0, 8), pl.ds(0, BM)]

        for b in range(_NB):
            fire_gathers(b, b)

        iota = lax.iota(jnp.int32, LANES)
        nr_vecs = [(iota + n0 * LANES) >> 3 for n0 in range(N // LANES)]
        n8_vecs = [(iota + n0 * LANES) & 7 for n0 in range(N // LANES)]

        def round_body(r, carry):
            for b in range(_NB):
                t = r * _NB + b
                wait_gathers(t, b)

                @pl.when(r > 0)
                def _():
                    pltpu.make_async_copy(
                        out_src(b), out_block(t - _NB), ssems[b]
                    ).wait()

                def bm_body(bm, _):
                    bm_vec = jnp.full((LANES,), 0, jnp.int32) + bm
                    for n0 in range(N // LANES):
                        sl = pl.ds(n0 * LANES, LANES)
                        acc = rows_v[b, 0, bm, sl]
                        for w in range(1, W):
                            acc = acc + rows_v[b, w, bm, sl]
                        plsc.store_scatter(
                            out_v.at[b],
                            [nr_vecs[n0], n8_vecs[n0], bm_vec],
                            acc * inv_w,
                        )
                    return _

                lax.fori_loop(0, BM, bm_body, 0)

                @pl.when(t + _NB < n_chunks)
                def _():
                    fire_gathers(t + _NB, b)

                pltpu.async_copy(out_src(b), out_block(t), ssems[b])

            return carry

        lax.fori_loop(0, n_rounds, round_body, 0)

        for b in range(_NB):
            pltpu.make_async_copy(
                out_src(b), out_block(n_chunks - _NB + b), ssems[b]
            ).wait()

    return sc_kernel


def kernel(indices, table):
    B, L, W = indices.shape
    V, N = table.shape
    try:
        info = plsc.get_sparse_core_info()
        NC, NS = info.num_cores, info.num_subcores
    except ValueError:
        NC, NS = 2, 16
    if indices.dtype != jnp.int32:
        indices = indices.astype(jnp.int32)
    tpad = jnp.pad(table, ((0, 0), (0, 128 - N)))
    it = jnp.transpose(indices, (1, 2, 0))
    it = it.reshape(L, W, B // 128, 128)
    it = jnp.transpose(it, (0, 2, 1, 3))
    it = it.reshape(L * B // 256, 8, 128)
    out = _make_sc_kernel(B, L, W, N, NC, NS)(it, tpad)
    o = jnp.transpose(out, (0, 1, 3, 2, 4))
    o = o.reshape(L, N, B)
    return jnp.transpose(o, (2, 0, 1))


# speedup vs baseline: 1.0054x; 1.0045x over previous
import functools

import jax
import jax.numpy as jnp
from jax import lax
from jax.experimental import pallas as pl
from jax.experimental.pallas import tpu as pltpu
from jax.experimental.pallas import tpu_sc as plsc

_NB = 2


def _make_tc_padder(V, N):
    BLK = 512

    def body(tt_ref, o_ref):
        t = jax.lax.dot_general(
            tt_ref[...],
            jnp.eye(N, dtype=jnp.float32),
            (((0,), (0,)), ((), ())),
            preferred_element_type=jnp.float32,
        )
        o_ref[...] = jnp.pad(t, ((0, 0), (0, 128 - N)))

    return pl.pallas_call(
        body,
        grid=(pl.cdiv(V, BLK),),
        in_specs=[pl.BlockSpec((N, BLK), lambda i: (0, i))],
        out_specs=pl.BlockSpec((BLK, 128), lambda i: (i, 0)),
        out_shape=jax.ShapeDtypeStruct((V, 128), jnp.float32),
    )


def _make_sc_kernel(B, L, W, N, NC, NS):
    NW = NC * NS
    BM = 64
    NBLK = B // 128
    n_chunks_tot = L * NBLK * 2
    n_chunks = n_chunks_tot // NW
    n_rounds = n_chunks // _NB
    NR = N // 8
    LANES = 16
    inv_w = 1.0 / W
    idx_rows_per_w = (n_chunks // 2) // 2

    mesh = plsc.VectorSubcoreMesh(
        core_axis_name="c", subcore_axis_name="s", num_cores=NC, num_subcores=NS
    )

    @functools.partial(
        pl.kernel,
        out_type=jax.ShapeDtypeStruct((L, NR, NBLK, 8, 128), jnp.float32),
        mesh=mesh,
        scratch_types=[
            pltpu.VMEM((idx_rows_per_w, 8, 128), jnp.int32),
            pltpu.VMEM((_NB, W, BM, 128), jnp.float32),
            pltpu.VMEM((_NB, NR, 8, 128), jnp.float32),
        ]
        + [pltpu.SemaphoreType.DMA] * (2 * _NB),
        compiler_params=pltpu.CompilerParams(
            needs_layout_passes=False, disable_bounds_checks=True
        ),
    )
    def sc_kernel(idx_hbm, table_hbm, out_hbm, idx_v, rows_v, out_v, *sems):
        gsems, ssems = sems[:_NB], sems[_NB:]
        wid = lax.axis_index("s") * NC + lax.axis_index("c")
        pltpu.sync_copy(
            idx_hbm.at[pl.ds(wid * idx_rows_per_w, idx_rows_per_w)], idx_v
        )

        def idx_slice(t, w):
            h = t & 1
            lcl = t >> 1
            p = lcl >> 1
            j = ((lcl & 1) << 2) + w
            return idx_v.at[p, j, pl.ds(h * BM, BM)]

        def fire_gathers(t, b):
            for w in range(W):
                pltpu.async_copy(
                    table_hbm.at[idx_slice(t, w)], rows_v.at[b, w], gsems[b]
                )

        def wait_gathers(t, b):
            for w in range(W):
                pltpu.make_async_copy(
                    table_hbm.at[idx_slice(t, w)], rows_v.at[b, w], gsems[b]
                ).wait()

        def out_block(t):
            g = wid * n_chunks + t
            l = g // (NBLK * 2)
            rem = g % (NBLK * 2)
            c = rem >> 1
            h = rem & 1
            return out_hbm.at[l, pl.ds(0, NR), c, pl.ds(0, 8), pl.ds(h * BM, BM)]

        def out_src(b):
            return out_v.at[b, pl.ds(0, NR), pl.ds(0, 8), pl.ds(0, BM)]

        for b in range(_NB):
            fire_gathers(b, b)

        iota = lax.iota(jnp.int32, LANES)
        nr_vecs = [(iota + n0 * LANES) >> 3 for n0 in range(N // LANES)]
        n8_vecs = [(iota + n0 * LANES) & 7 for n0 in range(N // LANES)]

        def round_body(r, carry):
            for b in range(_NB):
                t = r * _NB + b
                wait_gathers(t, b)

                @pl.when(r > 0)
                def _():
                    pltpu.make_async_copy(
                        out_src(b), out_block(t - _NB), ssems[b]
                    ).wait()

                def bm_body(bm2, _):
                    for u in range(2):
                        bm = bm2 * 2 + u
                        bm_vec = jnp.full((LANES,), 0, jnp.int32) + bm
                        for n0 in range(N // LANES):
                            sl = pl.ds(n0 * LANES, LANES)
                            acc = rows_v[b, 0, bm, sl]
                            for w in range(1, W):
                                acc = acc + rows_v[b, w, bm, sl]
                            plsc.store_scatter(
                                out_v.at[b],
                                [nr_vecs[n0], n8_vecs[n0], bm_vec],
                                acc * inv_w,
                            )
                    return _

                lax.fori_loop(0, BM // 2, bm_body, 0)

                @pl.when(t + _NB < n_chunks)
                def _():
                    fire_gathers(t + _NB, b)

                pltpu.async_copy(out_src(b), out_block(t), ssems[b])

            return carry

        lax.fori_loop(0, n_rounds, round_body, 0)

        for b in range(_NB):
            pltpu.make_async_copy(
                out_src(b), out_block(n_chunks - _NB + b), ssems[b]
            ).wait()

    return sc_kernel


def kernel(indices, table):
    B, L, W = indices.shape
    V, N = table.shape
    try:
        info = plsc.get_sparse_core_info()
        NC, NS = info.num_cores, info.num_subcores
    except ValueError:
        NC, NS = 2, 16
    if indices.dtype != jnp.int32:
        indices = indices.astype(jnp.int32)
    tpad = jnp.pad(table, ((0, 0), (0, 128 - N)))
    it = jnp.transpose(indices, (1, 2, 0))
    it = it.reshape(L, W, B // 128, 128)
    it = jnp.transpose(it, (0, 2, 1, 3))
    it = it.reshape(L * B // 256, 8, 128)
    out = _make_sc_kernel(B, L, W, N, NC, NS)(it, tpad)
    o = jnp.transpose(out, (0, 1, 3, 2, 4))
    o = o.reshape(L, N, B)
    return jnp.transpose(o, (2, 0, 1))


# speedup vs baseline: 1.0065x; 1.0011x over previous
import functools

import jax
import jax.numpy as jnp
from jax import lax
from jax.experimental import pallas as pl
from jax.experimental.pallas import tpu as pltpu
from jax.experimental.pallas import tpu_sc as plsc

_NB = 2


def _make_sc_kernel(B, L, W, N, NC, NS):
    NW = NC * NS
    BM = 64
    NBLK = B // 128
    n_chunks_tot = L * NBLK * 2
    n_chunks = n_chunks_tot // NW
    n_rounds = n_chunks // _NB
    NR = N // 8
    LANES = 16
    inv_w = 1.0 / W
    idx_rows_per_w = (n_chunks // 2) // 2

    mesh = plsc.VectorSubcoreMesh(
        core_axis_name="c", subcore_axis_name="s", num_cores=NC, num_subcores=NS
    )

    @functools.partial(
        pl.kernel,
        out_type=jax.ShapeDtypeStruct((L, NR, NBLK, 8, 128), jnp.float32),
        mesh=mesh,
        scratch_types=[
            pltpu.VMEM((idx_rows_per_w, 8, 128), jnp.int32),
            pltpu.VMEM((_NB, W, BM, 128), jnp.float32),
            pltpu.VMEM((_NB, NR, 8, 128), jnp.float32),
        ]
        + [pltpu.SemaphoreType.DMA] * (2 * _NB),
        compiler_params=pltpu.CompilerParams(
            needs_layout_passes=False, disable_bounds_checks=True
        ),
    )
    def sc_kernel(idx_hbm, table_hbm, out_hbm, idx_v, rows_v, out_v, *sems):
        gsems, ssems = sems[:_NB], sems[_NB:]
        wid = lax.axis_index("s") * NC + lax.axis_index("c")
        pltpu.sync_copy(
            idx_hbm.at[pl.ds(wid * idx_rows_per_w, idx_rows_per_w)], idx_v
        )

        def idx_slice(t, w):
            h = t & 1
            lcl = t >> 1
            p = lcl >> 1
            j = ((lcl & 1) << 2) + w
            return idx_v.at[p, j, pl.ds(h * BM, BM)]

        def fire_gathers(t, b):
            for w in range(W):
                pltpu.async_copy(
                    table_hbm.at[idx_slice(t, w)], rows_v.at[b, w], gsems[b]
                )

        def wait_gathers(t, b):
            for w in range(W):
                pltpu.make_async_copy(
                    table_hbm.at[idx_slice(t, w)], rows_v.at[b, w], gsems[b]
                ).wait()

        def out_block(t):
            g = wid * n_chunks + t
            l = g // (NBLK * 2)
            rem = g % (NBLK * 2)
            c = rem >> 1
            h = rem & 1
            return out_hbm.at[l, pl.ds(0, NR), c, pl.ds(0, 8), pl.ds(h * BM, BM)]

        def out_src(b):
            return out_v.at[b, pl.ds(0, NR), pl.ds(0, 8), pl.ds(0, BM)]

        for b in range(_NB):
            fire_gathers(b, b)

        iota = lax.iota(jnp.int32, LANES)
        nr_vecs = [(iota + n0 * LANES) >> 3 for n0 in range(N // LANES)]
        n8_vecs = [(iota + n0 * LANES) & 7 for n0 in range(N // LANES)]

        def round_body(r, carry):
            for b in range(_NB):
                t = r * _NB + b
                wait_gathers(t, b)

                @pl.when(r > 0)
                def _():
                    pltpu.make_async_copy(
                        out_src(b), out_block(t - _NB), ssems[b]
                    ).wait()

                def bm_body(bm2, _):
                    for u in range(2):
                        bm = bm2 * 2 + u
                        bm_vec = jnp.full((LANES,), 0, jnp.int32) + bm
                        for n0 in range(N // LANES):
                            sl = pl.ds(n0 * LANES, LANES)
                            acc = rows_v[b, 0, bm, sl]
                            for w in range(1, W):
                                acc = acc + rows_v[b, w, bm, sl]
                            plsc.store_scatter(
                                out_v.at[b],
                                [nr_vecs[n0], n8_vecs[n0], bm_vec],
                                acc * inv_w,
                            )
                    return _

                lax.fori_loop(0, BM // 2, bm_body, 0)

                @pl.when(t + _NB < n_chunks)
                def _():
                    fire_gathers(t + _NB, b)

                pltpu.async_copy(out_src(b), out_block(t), ssems[b])

            return carry

        lax.fori_loop(0, n_rounds, round_body, 0)

        for b in range(_NB):
            pltpu.make_async_copy(
                out_src(b), out_block(n_chunks - _NB + b), ssems[b]
            ).wait()

    return sc_kernel


def kernel(indices, table):
    B, L, W = indices.shape
    V, N = table.shape
    try:
        info = plsc.get_sparse_core_info()
        NC, NS = info.num_cores, info.num_subcores
    except ValueError:
        NC, NS = 2, 16
    if indices.dtype != jnp.int32:
        indices = indices.astype(jnp.int32)
    tpad = jnp.pad(table, ((0, 0), (0, 128 - N)))
    it = jnp.transpose(indices, (1, 2, 0))
    it = it.reshape(L, W, B // 128, 128)
    it = jnp.transpose(it, (0, 2, 1, 3))
    it = it.reshape(L * B // 256, 8, 128)
    out = _make_sc_kernel(B, L, W, N, NC, NS)(it, tpad)
    o = jnp.transpose(out, (0, 1, 3, 2, 4))
    o = o.reshape(L, N, B)
    return jnp.transpose(o, (2, 0, 1))
